# baseline (speedup 1.0000x reference)
import math

import jax
import jax.numpy as jnp
from jax.experimental import pallas as pl
from jax.experimental.pallas import tpu as pltpu

N_MOL = 100
HIDDEN = 128
NFILT = 128
N_INT = 6
NG = 50
CUTOFF = 0.25
CH = 8
SZ = 128

_DELTA = CUTOFF / (NG - 1)
_COEFF = -0.5 / _DELTA ** 2
_LN2 = math.log(2.0)
_HI = jax.lax.Precision.HIGHEST
_PR = jax.lax.Precision.DEFAULT


def _ssp(x):
    return jnp.maximum(x, 0.0) + jnp.log1p(jnp.exp(-jnp.abs(x))) - _LN2


TD = 128


def _body(sinfo, pos_ref, bcol_ref, zcol_ref, emb_ref,
          W1_ref, b1_ref, W2_ref, b2_ref, lin1_ref, lin2_ref, lin2b_ref,
          intW_ref, intb_ref, o1W_ref, o1b_ref, o2W_ref, o2b_ref,
          out_ref, h_ref, xl_ref, agg_ref):
    i = pl.program_id(0)
    t = pl.program_id(1)
    NP = h_ref.shape[0]
    F = NFILT

    @pl.when(jnp.logical_and(i == 0, t == 0))
    def _init():
        z = zcol_ref[...]
        lane = jax.lax.broadcasted_iota(jnp.int32, (NP, 128), 1)
        onehot = (z == lane).astype(jnp.float32)
        h_ref[...] = jnp.dot(onehot, emb_ref[...], precision=_PR,
                             preferred_element_type=jnp.float32)
        out_ref[...] = jnp.zeros_like(out_ref)
        xl_ref[pl.ds(NP, SZ), :] = jnp.zeros((SZ, NFILT), jnp.float32)

    @pl.when(t == 0)
    def _xl():
        xl_ref[pl.ds(0, NP), :] = jnp.dot(
            h_ref[...], lin1_ref[i], precision=_PR,
            preferred_element_type=jnp.float32)

    dst0 = t * TD
    W1 = W1_ref[i]
    b1 = b1_ref[i]
    W2 = W2_ref[i]
    b2 = b2_ref[i]

    def chunk_body(cc, _):
        c = t * (TD // CH) + cc
        ch0 = dst0 + cc * CH
        pd = pos_ref[pl.ds(ch0, CH), :]
        dst_ids = ch0 + jax.lax.broadcasted_iota(jnp.int32, (CH, 1), 0)
        s0 = sinfo[0, c]
        ns = sinfo[1, c]

        def src_body(k, agg):
            sb = s0 + k * SZ
            psT = pos_ref[pl.ds(sb, SZ), :].T
            d2 = jnp.zeros((CH, SZ), jnp.float32)
            for cd in range(3):
                diff = pd[:, cd:cd + 1] - psT[cd:cd + 1, :]
                d2 = d2 + diff * diff
            dw = pd[:, 3:4] - psT[3:4, :]
            src_ids = sb + jax.lax.broadcasted_iota(jnp.int32, (1, SZ), 1)
            mask = (d2 + dw * dw < CUTOFF * CUTOFF) & (dst_ids != src_ids)
            d = jnp.sqrt(d2)
            cm = 0.5 * (jnp.cos(d * (math.pi / CUTOFF)) + 1.0)
            scale = jnp.where(mask, cm, 0.0)
            off = jax.lax.broadcasted_iota(
                jnp.int32, (1, 1, NG), 2).astype(jnp.float32) * _DELTA
            rbf = jnp.exp(_COEFF * (d[:, :, None] - off) ** 2)
            A = jnp.dot(rbf.reshape(CH * SZ, NG), W1, precision=_PR,
                        preferred_element_type=jnp.float32) + b1
            A = _ssp(A)
            B = jnp.dot(A, W2, precision=_PR,
                        preferred_element_type=jnp.float32) + b2
            xs = xl_ref[pl.ds(sb, SZ), :]
            msg = B.reshape(CH, SZ, F) * scale[:, :, None] * xs[None, :, :]
            return agg + jnp.sum(msg, axis=1)

        aggc = jax.lax.fori_loop(0, ns, src_body,
                                 jnp.zeros((CH, F), jnp.float32))
        agg_ref[pl.ds(cc * CH, CH), :] = aggc
        return 0

    jax.lax.fori_loop(0, TD // CH, chunk_body, 0)
    bd = bcol_ref[pl.ds(dst0, TD), :]
    xc = jnp.dot(agg_ref[...], lin2_ref[i], precision=_PR,
                 preferred_element_type=jnp.float32) + lin2b_ref[i]
    v = jnp.dot(_ssp(xc), intW_ref[i], precision=_PR,
                preferred_element_type=jnp.float32) + intb_ref[i]
    h_new = h_ref[pl.ds(dst0, TD), :] + v
    h_ref[pl.ds(dst0, TD), :] = h_new

    @pl.when(i == N_INT - 1)
    def _readout():
        u = _ssp(jnp.dot(h_new, o1W_ref[...], precision=_PR,
                         preferred_element_type=jnp.float32) + o1b_ref[...])
        h2 = jnp.dot(u, o2W_ref[...], precision=_PR,
                     preferred_element_type=jnp.float32) + o2b_ref[...]
        lane = jax.lax.broadcasted_iota(jnp.int32, (TD, 128), 1)
        onehot = (bd == lane).astype(jnp.float32)
        contrib = jax.lax.dot_general(
            h2, onehot, (((0,), (0,)), ((), ())), precision=_PR,
            preferred_element_type=jnp.float32)
        out_ref[...] += contrib


def _full(shape):
    nd = len(shape)
    return pl.BlockSpec(shape, lambda i, t, s: (0,) * nd)


@jax.jit
def _run(z, pos, batch, emb, mlp_W1, mlp_b1, mlp_W2, mlp_b2, conv_lin1_W,
         conv_lin2_W, conv_lin2_b, int_lin_W, int_lin_b, out1_W, out1_b,
         out2_W, out2_b):
    f32 = jnp.float32
    n = pos.shape[0]
    T = (n + TD - 1) // TD
    NP = T * TD
    NC = NP // CH
    P_ = NP - n

    pos = pos.astype(f32)
    padv = 2.0 + 2.0 * jnp.arange(P_, dtype=f32)
    z_p = jnp.concatenate([z.astype(jnp.int32), jnp.zeros((P_,), jnp.int32)])
    b32 = batch.astype(jnp.int32)
    batch_p = jnp.concatenate([b32, jnp.full((P_,), 127, jnp.int32)])
    tailpos = 1e6 + 1e3 * jnp.arange(SZ, dtype=f32)
    pos_mid = jnp.concatenate([pos, jnp.stack([padv] * 3, axis=1)], axis=0)
    w4 = jnp.concatenate(
        [10.0 * batch_p.astype(f32), jnp.full((SZ,), 1260.0, f32)])
    pos4 = jnp.concatenate(
        [jnp.concatenate([pos_mid, jnp.stack([tailpos] * 3, axis=1)], axis=0),
         w4[:, None]], axis=1)
    bcol = batch_p[:, None]
    zcol = z_p[:, None]
    emb_p = jnp.zeros((128, HIDDEN), f32).at[:emb.shape[0]].set(emb.astype(f32))

    firsts = batch_p[0::CH][:NC]
    lasts = batch_p[CH - 1::CH][:NC]
    lo = jnp.searchsorted(b32, firsts, side='left')
    hi = jnp.searchsorted(b32, lasts, side='right')
    lo8 = ((lo // 8) * 8).astype(jnp.int32)
    ntl = ((hi - lo8 + SZ - 1) // SZ).astype(jnp.int32)
    sinfo = jnp.stack([lo8, jnp.maximum(ntl, 1)])

    b1r = mlp_b1.reshape(N_INT, 1, NFILT).astype(f32)
    b2r = mlp_b2.reshape(N_INT, 1, NFILT).astype(f32)
    lin2b = conv_lin2_b.reshape(N_INT, 1, HIDDEN).astype(f32)
    intb = int_lin_b.reshape(N_INT, 1, HIDDEN).astype(f32)
    o1b = out1_b.reshape(1, -1).astype(f32)
    o2b = out2_b.reshape(1, 1).astype(f32)

    inputs = (pos4, bcol, zcol, emb_p,
              mlp_W1.astype(f32), b1r, mlp_W2.astype(f32), b2r,
              conv_lin1_W.astype(f32), conv_lin2_W.astype(f32), lin2b,
              int_lin_W.astype(f32), intb, out1_W.astype(f32), o1b,
              out2_W.astype(f32), o2b)

    grid_spec = pltpu.PrefetchScalarGridSpec(
        num_scalar_prefetch=1,
        grid=(N_INT, T),
        in_specs=[_full(a.shape) for a in inputs],
        out_specs=pl.BlockSpec((1, 128), lambda i, t, s: (0, 0)),
        scratch_shapes=[pltpu.VMEM((NP, HIDDEN), f32),
                        pltpu.VMEM((NP + SZ, NFILT), f32),
                        pltpu.VMEM((TD, NFILT), f32)],
    )
    out = pl.pallas_call(
        _body,
        grid_spec=grid_spec,
        out_shape=jax.ShapeDtypeStruct((1, 128), f32),
        compiler_params=pltpu.CompilerParams(
            vmem_limit_bytes=100 * 1024 * 1024),
    )(sinfo, *inputs)
    return out[0, :N_MOL]


def kernel(z, pos, batch, emb, mlp_W1, mlp_b1, mlp_W2, mlp_b2, conv_lin1_W,
           conv_lin2_W, conv_lin2_b, int_lin_W, int_lin_b, out1_W, out1_b,
           out2_W, out2_b):
    return _run(z, pos, batch, emb, mlp_W1, mlp_b1, mlp_W2, mlp_b2,
                conv_lin1_W, conv_lin2_W, conv_lin2_b, int_lin_W, int_lin_b,
                out1_W, out1_b, out2_W, out2_b)


# speedup vs baseline: 1.1708x; 1.1708x over previous
import math

import jax
import jax.numpy as jnp
from jax.experimental import pallas as pl
from jax.experimental.pallas import tpu as pltpu

N_MOL = 100
HIDDEN = 128
NFILT = 128
N_INT = 6
NG = 50
CUTOFF = 0.25
CH = 16
SZ = 128

_DELTA = CUTOFF / (NG - 1)
_COEFF = -0.5 / _DELTA ** 2
_LN2 = math.log(2.0)
_HI = jax.lax.Precision.HIGHEST
_PR = jax.lax.Precision.DEFAULT


def _ssp(x):
    return jnp.maximum(x, 0.0) + jnp.log1p(jnp.exp(-jnp.abs(x))) - _LN2


TD = 256


def _body(sinfo, pos_ref, bcol_ref, zcol_ref, emb_ref,
          W1_ref, b1_ref, W2_ref, b2_ref, lin1_ref, lin2_ref, lin2b_ref,
          intW_ref, intb_ref, o1W_ref, o1b_ref, o2W_ref, o2b_ref,
          out_ref, h_ref, xl_ref, agg_ref):
    i = pl.program_id(0)
    t = pl.program_id(1)
    NP = h_ref.shape[0]
    F = NFILT

    @pl.when(jnp.logical_and(i == 0, t == 0))
    def _init():
        z = zcol_ref[...]
        lane = jax.lax.broadcasted_iota(jnp.int32, (NP, 128), 1)
        onehot = (z == lane).astype(jnp.float32)
        h_ref[...] = jnp.dot(onehot, emb_ref[...], precision=_PR,
                             preferred_element_type=jnp.float32)
        out_ref[...] = jnp.zeros_like(out_ref)
        xl_ref[pl.ds(NP, SZ), :] = jnp.zeros((SZ, NFILT), jnp.float32)

    @pl.when(t == 0)
    def _xl():
        xl_ref[pl.ds(0, NP), :] = jnp.dot(
            h_ref[...], lin1_ref[i], precision=_PR,
            preferred_element_type=jnp.float32)

    dst0 = t * TD
    W1 = W1_ref[i]
    b1 = b1_ref[i]
    W2 = W2_ref[i]
    b2 = b2_ref[i]

    def chunk_body(cc, _):
        c = t * (TD // CH) + cc
        ch0 = dst0 + cc * CH
        pd = pos_ref[pl.ds(ch0, CH), :]
        dst_ids = ch0 + jax.lax.broadcasted_iota(jnp.int32, (CH, 1), 0)
        s0 = sinfo[0, c]
        ns = sinfo[1, c]

        def src_body(k, agg):
            sb = s0 + k * SZ
            psT = pos_ref[pl.ds(sb, SZ), :].T
            d2 = jnp.zeros((CH, SZ), jnp.float32)
            for cd in range(3):
                diff = pd[:, cd:cd + 1] - psT[cd:cd + 1, :]
                d2 = d2 + diff * diff
            dw = pd[:, 3:4] - psT[3:4, :]
            src_ids = sb + jax.lax.broadcasted_iota(jnp.int32, (1, SZ), 1)
            mask = (d2 + dw * dw < CUTOFF * CUTOFF) & (dst_ids != src_ids)
            d = jnp.sqrt(d2)
            cm = 0.5 * (jnp.cos(d * (math.pi / CUTOFF)) + 1.0)
            scale = jnp.where(mask, cm, 0.0)
            off = jax.lax.broadcasted_iota(
                jnp.int32, (1, 1, NG), 2).astype(jnp.float32) * _DELTA
            rbf = jnp.exp(_COEFF * (d[:, :, None] - off) ** 2)
            A = jnp.dot(rbf.reshape(CH * SZ, NG), W1, precision=_PR,
                        preferred_element_type=jnp.float32) + b1
            A = _ssp(A)
            B = jnp.dot(A, W2, precision=_PR,
                        preferred_element_type=jnp.float32) + b2
            xs = xl_ref[pl.ds(sb, SZ), :]
            msg = B.reshape(CH, SZ, F) * scale[:, :, None] * xs[None, :, :]
            return agg + jnp.sum(msg, axis=1)

        aggc = jax.lax.fori_loop(0, ns, src_body,
                                 jnp.zeros((CH, F), jnp.float32))
        agg_ref[pl.ds(cc * CH, CH), :] = aggc
        return 0

    jax.lax.fori_loop(0, TD // CH, chunk_body, 0)
    bd = bcol_ref[pl.ds(dst0, TD), :]
    xc = jnp.dot(agg_ref[...], lin2_ref[i], precision=_PR,
                 preferred_element_type=jnp.float32) + lin2b_ref[i]
    v = jnp.dot(_ssp(xc), intW_ref[i], precision=_PR,
                preferred_element_type=jnp.float32) + intb_ref[i]
    h_new = h_ref[pl.ds(dst0, TD), :] + v
    h_ref[pl.ds(dst0, TD), :] = h_new

    @pl.when(i == N_INT - 1)
    def _readout():
        u = _ssp(jnp.dot(h_new, o1W_ref[...], precision=_PR,
                         preferred_element_type=jnp.float32) + o1b_ref[...])
        h2 = jnp.dot(u, o2W_ref[...], precision=_PR,
                     preferred_element_type=jnp.float32) + o2b_ref[...]
        lane = jax.lax.broadcasted_iota(jnp.int32, (TD, 128), 1)
        onehot = (bd == lane).astype(jnp.float32)
        contrib = jax.lax.dot_general(
            h2, onehot, (((0,), (0,)), ((), ())), precision=_PR,
            preferred_element_type=jnp.float32)
        out_ref[...] += contrib


def _full(shape):
    nd = len(shape)
    return pl.BlockSpec(shape, lambda i, t, s: (0,) * nd)


@jax.jit
def _run(z, pos, batch, emb, mlp_W1, mlp_b1, mlp_W2, mlp_b2, conv_lin1_W,
         conv_lin2_W, conv_lin2_b, int_lin_W, int_lin_b, out1_W, out1_b,
         out2_W, out2_b):
    f32 = jnp.float32
    n = pos.shape[0]
    T = (n + TD - 1) // TD
    NP = T * TD
    NC = NP // CH
    P_ = NP - n

    pos = pos.astype(f32)
    padv = 2.0 + 2.0 * jnp.arange(P_, dtype=f32)
    z_p = jnp.concatenate([z.astype(jnp.int32), jnp.zeros((P_,), jnp.int32)])
    b32 = batch.astype(jnp.int32)
    batch_p = jnp.concatenate([b32, jnp.full((P_,), 127, jnp.int32)])
    tailpos = 1e6 + 1e3 * jnp.arange(SZ, dtype=f32)
    pos_mid = jnp.concatenate([pos, jnp.stack([padv] * 3, axis=1)], axis=0)
    w4 = jnp.concatenate(
        [10.0 * batch_p.astype(f32), jnp.full((SZ,), 1260.0, f32)])
    pos4 = jnp.concatenate(
        [jnp.concatenate([pos_mid, jnp.stack([tailpos] * 3, axis=1)], axis=0),
         w4[:, None]], axis=1)
    bcol = batch_p[:, None]
    zcol = z_p[:, None]
    emb_p = jnp.zeros((128, HIDDEN), f32).at[:emb.shape[0]].set(emb.astype(f32))

    firsts = batch_p[0::CH][:NC]
    lasts = batch_p[CH - 1::CH][:NC]
    lo = jnp.searchsorted(b32, firsts, side='left')
    hi = jnp.searchsorted(b32, lasts, side='right')
    lo8 = ((lo // 8) * 8).astype(jnp.int32)
    ntl = ((hi - lo8 + SZ - 1) // SZ).astype(jnp.int32)
    sinfo = jnp.stack([lo8, jnp.maximum(ntl, 1)])

    b1r = mlp_b1.reshape(N_INT, 1, NFILT).astype(f32)
    b2r = mlp_b2.reshape(N_INT, 1, NFILT).astype(f32)
    lin2b = conv_lin2_b.reshape(N_INT, 1, HIDDEN).astype(f32)
    intb = int_lin_b.reshape(N_INT, 1, HIDDEN).astype(f32)
    o1b = out1_b.reshape(1, -1).astype(f32)
    o2b = out2_b.reshape(1, 1).astype(f32)

    inputs = (pos4, bcol, zcol, emb_p,
              mlp_W1.astype(f32), b1r, mlp_W2.astype(f32), b2r,
              conv_lin1_W.astype(f32), conv_lin2_W.astype(f32), lin2b,
              int_lin_W.astype(f32), intb, out1_W.astype(f32), o1b,
              out2_W.astype(f32), o2b)

    grid_spec = pltpu.PrefetchScalarGridSpec(
        num_scalar_prefetch=1,
        grid=(N_INT, T),
        in_specs=[_full(a.shape) for a in inputs],
        out_specs=pl.BlockSpec((1, 128), lambda i, t, s: (0, 0)),
        scratch_shapes=[pltpu.VMEM((NP, HIDDEN), f32),
                        pltpu.VMEM((NP + SZ, NFILT), f32),
                        pltpu.VMEM((TD, NFILT), f32)],
    )
    out = pl.pallas_call(
        _body,
        grid_spec=grid_spec,
        out_shape=jax.ShapeDtypeStruct((1, 128), f32),
        compiler_params=pltpu.CompilerParams(
            vmem_limit_bytes=100 * 1024 * 1024),
    )(sinfo, *inputs)
    return out[0, :N_MOL]


def kernel(z, pos, batch, emb, mlp_W1, mlp_b1, mlp_W2, mlp_b2, conv_lin1_W,
           conv_lin2_W, conv_lin2_b, int_lin_W, int_lin_b, out1_W, out1_b,
           out2_W, out2_b):
    return _run(z, pos, batch, emb, mlp_W1, mlp_b1, mlp_W2, mlp_b2,
                conv_lin1_W, conv_lin2_W, conv_lin2_b, int_lin_W, int_lin_b,
                out1_W, out1_b, out2_W, out2_b)
